# baseline (speedup 1.0000x reference)
import functools
import jax
import jax.numpy as jnp
from jax import lax
from jax.experimental import pallas as pl
from jax.experimental.pallas import tpu as pltpu
from jax.experimental.pallas import tpu_sc as plsc

_HIDDEN = 2048
_K = 16
_NBUF = 2


def _make_sc_kernel(n_total, nc, ns):
    nw = nc * ns
    b_per_w = n_total // nw
    ch = b_per_w // _K
    pairs = ch // _NBUF

    mesh = plsc.VectorSubcoreMesh(
        core_axis_name="c", subcore_axis_name="s", num_cores=nc, num_subcores=ns
    )

    @functools.partial(
        pl.kernel,
        out_type=jax.ShapeDtypeStruct((n_total, _HIDDEN), jnp.float32),
        mesh=mesh,
        scratch_types=[
            pltpu.VMEM((ch, _K), jnp.int32),
            pltpu.VMEM((_K, _HIDDEN), jnp.float32),
            pltpu.VMEM((_K, _HIDDEN), jnp.float32),
            pltpu.SemaphoreType.DMA,
            pltpu.SemaphoreType.DMA,
        ],
    )
    def k(table_hbm, idx_hbm, out_hbm, idx_v, rows0, rows1, sem0, sem1):
        wid = lax.axis_index("s") * nc + lax.axis_index("c")
        base = wid * b_per_w
        pltpu.sync_copy(idx_hbm.at[wid], idx_v)

        bufs = (rows0, rows1)
        sems = (sem0, sem1)

        def gather(g, nb):
            return pltpu.async_copy(table_hbm.at[idx_v.at[g]], bufs[nb], sems[nb])

        gather(0, 0)

        def step(p, _):
            g0 = p * _NBUF
            gather(g0 + 1, 1)
            pltpu.make_async_copy(table_hbm.at[idx_v.at[g0]], bufs[0], sems[0]).wait()
            pltpu.sync_copy(bufs[0], out_hbm.at[pl.ds(base + g0 * _K, _K)])
            @pl.when(p < pairs - 1)
            def _():
                gather(g0 + 2, 0)
            pltpu.make_async_copy(table_hbm.at[idx_v.at[g0 + 1]], bufs[1], sems[1]).wait()
            pltpu.sync_copy(bufs[1], out_hbm.at[pl.ds(base + (g0 + 1) * _K, _K)])
            return 0

        lax.fori_loop(0, pairs, step, 0)

    return k


def kernel(modality_ids, table):
    b, s = modality_ids.shape
    n = b * s
    nc, ns = 2, 16
    nw = nc * ns
    ids3 = modality_ids.reshape(nw, (n // nw) // _K, _K).astype(jnp.int32)
    k = _make_sc_kernel(n, nc, ns)
    out = k(table, ids3)
    return out.reshape(b, s, _HIDDEN)


# speedup vs baseline: 7.6173x; 7.6173x over previous
import functools
import jax
import jax.numpy as jnp
from jax import lax
from jax.experimental import pallas as pl
from jax.experimental.pallas import tpu as pltpu
from jax.experimental.pallas import tpu_sc as plsc

_HIDDEN = 2048
_K = 16


def _make_sc_kernel(n_total, nc, ns):
    nw = nc * ns
    b_per_w = n_total // nw
    ch = b_per_w // _K

    mesh = plsc.VectorSubcoreMesh(
        core_axis_name="c", subcore_axis_name="s", num_cores=nc, num_subcores=ns
    )

    @functools.partial(
        pl.kernel,
        out_type=jax.ShapeDtypeStruct((n_total, _HIDDEN), jnp.float32),
        mesh=mesh,
        scratch_types=[
            pltpu.VMEM((3, _HIDDEN), jnp.float32),
            pltpu.VMEM((ch, _K), jnp.int32),
            pltpu.SemaphoreType.DMA,
        ],
    )
    def k(table_hbm, idx_hbm, out_hbm, table_v, idx_v, sem):
        wid = lax.axis_index("s") * nc + lax.axis_index("c")
        base = wid * b_per_w
        pltpu.sync_copy(table_hbm, table_v)
        pltpu.sync_copy(idx_hbm.at[wid], idx_v)

        def fire(g):
            idxvec = idx_v[g, :]
            for r in range(_K):
                rowid = idxvec[r]
                pltpu.async_copy(
                    table_v.at[pl.ds(rowid, 1)],
                    out_hbm.at[pl.ds(base + g * _K + r, 1)],
                    sem,
                )

        def drain(count):
            for _ in range(count):
                pltpu.make_async_copy(
                    table_v.at[pl.ds(0, 1)], out_hbm.at[pl.ds(base, 1)], sem
                ).wait()

        def step(g, _):
            fire(g)

            @pl.when(g >= 2)
            def _():
                drain(_K)

            return 0

        lax.fori_loop(0, ch, step, 0)
        drain(2 * _K)

    return k


def kernel(modality_ids, table):
    b, s = modality_ids.shape
    n = b * s
    nc, ns = 2, 16
    nw = nc * ns
    ids3 = modality_ids.reshape(nw, (n // nw) // _K, _K).astype(jnp.int32)
    k = _make_sc_kernel(n, nc, ns)
    out = k(table, ids3)
    return out.reshape(b, s, _HIDDEN)
